# baseline (speedup 1.0000x reference)
import functools

import jax
import jax.numpy as jnp
import numpy as np
from jax import lax
from jax.experimental import pallas as pl
from jax.experimental.pallas import tpu as pltpu
from jax.experimental.pallas import tpu_sc as plsc

_PATCH = 16
_SIDELEN = 32
_T = 16


def _threefry2x32(k1, k2, x0, x1):
    x0 = x0.astype(np.uint32).copy()
    x1 = x1.astype(np.uint32).copy()
    rot = ((13, 15, 26, 6), (17, 29, 16, 24))
    ks = (np.uint32(k1), np.uint32(k2),
          np.uint32(np.uint32(k1) ^ np.uint32(k2) ^ np.uint32(0x1BD11BDA)))
    x0 += ks[0]
    x1 += ks[1]
    for i in range(5):
        for r in rot[i % 2]:
            x0 += x1
            x1 = (x1 << np.uint32(r)) | (x1 >> np.uint32(32 - r))
            x1 ^= x0
        x0 += ks[(i + 1) % 3]
        x1 += ks[(i + 2) % 3] + np.uint32(i + 1)
    return x0, x1


def _np_split(key, num):
    b1, b2 = _threefry2x32(key[0], key[1],
                           np.zeros(num, np.uint32), np.arange(num, dtype=np.uint32))
    return np.stack([b1, b2], axis=1)


def _np_permutation(key, n):
    ks = _np_split(key, 2)
    b1, b2 = _threefry2x32(ks[1][0], ks[1][1],
                           np.zeros(n, np.uint32), np.arange(n, dtype=np.uint32))
    return np.argsort(b1 ^ b2, kind="stable").astype(np.int32)


def _build_row_mask() -> np.ndarray:
    num_patches = _SIDELEN * _SIDELEN
    num_masked = num_patches // 2
    keys = _np_split(np.array([0, 42], np.uint32), _T)
    idx = np.stack([_np_permutation(keys[t], num_patches)[:num_masked]
                    for t in range(_T)])
    h = idx % _SIDELEN
    w = idx // _SIDELEN
    pm = np.ones((_T, _SIDELEN, _SIDELEN), np.float32)
    pm[np.arange(_T)[:, None], h, w] = 0.0
    return np.repeat(pm, _PATCH, axis=2)


_ROW_MASK = _build_row_mask()
_MASK2D = np.tile(_ROW_MASK.reshape(_T * _SIDELEN, 512), (3, 1))

_NC, _NS = 2, 16
_NW = _NC * _NS
_BANDS = 3 * _T * _SIDELEN
_PXROWS = _BANDS * _PATCH
_BPW = _BANDS // _NW
_BCHUNK = 2
_PCHUNK = _BCHUNK * _PATCH
_NCHUNKS = _BPW // _BCHUNK
_NBUF = 6
_AHEAD = _NBUF - 1


def _sc_body(f_hbm, m_hbm, o_hbm, b0, b1, b2, b3, b4, b5, mbuf,
             si0, si1, si2, si3, si4, si5, so0, so1, so2, so3, so4, so5):
    bufs = (b0, b1, b2, b3, b4, b5)
    isems = (si0, si1, si2, si3, si4, si5)
    osems = (so0, so1, so2, so3, so4, so5)
    wid = lax.axis_index("s") * _NC + lax.axis_index("c")
    band0 = wid * _BPW
    px0 = band0 * _PATCH
    pltpu.sync_copy(m_hbm.at[pl.ds(band0, _BPW)], mbuf)

    for b in range(_AHEAD):
        pltpu.async_copy(f_hbm.at[pl.ds(px0 + b * _PCHUNK, _PCHUNK)],
                         bufs[b], isems[b])

    def compute_chunk(k, s):
        for r in range(_BCHUNK):
            mrow = k * _BCHUNK + r
            mvecs = [mbuf[mrow, pl.ds(v * 16, 16)] for v in range(_SIDELEN)]

            @plsc.parallel_loop(0, _PATCH, unroll=1)
            def px_body(py, _mvecs=mvecs, _r=r, _s=s):
                row = _r * _PATCH + py
                for v in range(_SIDELEN):
                    sl = (row, pl.ds(v * 16, 16))
                    bufs[_s][sl] = bufs[_s][sl] * _mvecs[v]

    def group_body(g, carry):
        for b in range(_NBUF):
            k = g * _NBUF + b
            s = b
            s2 = (b + _AHEAD) % _NBUF
            rs = px0 + k * _PCHUNK
            pltpu.make_async_copy(f_hbm.at[pl.ds(rs, _PCHUNK)],
                                  bufs[s], isems[s]).wait()
            compute_chunk(k, s)
            pltpu.async_copy(bufs[s], o_hbm.at[pl.ds(rs, _PCHUNK)], osems[s])

            @pl.when(k + _AHEAD < _NCHUNKS)
            def _(k=k, s2=s2):
                @pl.when(k >= 1)
                def _():
                    prs = px0 + (k - 1) * _PCHUNK
                    pltpu.make_async_copy(bufs[s2], o_hbm.at[pl.ds(prs, _PCHUNK)],
                                          osems[s2]).wait()

                nrs = px0 + (k + _AHEAD) * _PCHUNK
                pltpu.async_copy(f_hbm.at[pl.ds(nrs, _PCHUNK)], bufs[s2], isems[s2])
        return carry

    lax.fori_loop(0, _NCHUNKS // _NBUF, group_body, 0)

    for k in range(_NCHUNKS - _AHEAD - 1, _NCHUNKS):
        s = k % _NBUF
        lrs = px0 + k * _PCHUNK
        pltpu.make_async_copy(bufs[s], o_hbm.at[pl.ds(lrs, _PCHUNK)],
                              osems[s]).wait()


@functools.partial(
    pl.kernel,
    out_type=jax.ShapeDtypeStruct((_PXROWS, 512), jnp.float32),
    mesh=plsc.VectorSubcoreMesh(core_axis_name="c", subcore_axis_name="s"),
    scratch_types=[
        pltpu.VMEM((_PCHUNK, 512), jnp.float32),
        pltpu.VMEM((_PCHUNK, 512), jnp.float32),
        pltpu.VMEM((_PCHUNK, 512), jnp.float32),
        pltpu.VMEM((_PCHUNK, 512), jnp.float32),
        pltpu.VMEM((_PCHUNK, 512), jnp.float32),
        pltpu.VMEM((_PCHUNK, 512), jnp.float32),
        pltpu.VMEM((_BPW, 512), jnp.float32),
        pltpu.SemaphoreType.DMA,
        pltpu.SemaphoreType.DMA,
        pltpu.SemaphoreType.DMA,
        pltpu.SemaphoreType.DMA,
        pltpu.SemaphoreType.DMA,
        pltpu.SemaphoreType.DMA,
        pltpu.SemaphoreType.DMA,
        pltpu.SemaphoreType.DMA,
        pltpu.SemaphoreType.DMA,
        pltpu.SemaphoreType.DMA,
        pltpu.SemaphoreType.DMA,
        pltpu.SemaphoreType.DMA,
    ],
)
def _sc_mask(f_hbm, m_hbm, o_hbm, b0, b1, b2, b3, b4, b5, mbuf,
             si0, si1, si2, si3, si4, si5, so0, so1, so2, so3, so4, so5):
    _sc_body(f_hbm, m_hbm, o_hbm, b0, b1, b2, b3, b4, b5, mbuf,
             si0, si1, si2, si3, si4, si5, so0, so1, so2, so3, so4, so5)


def kernel(frames):
    C, T, H, W = frames.shape
    f2 = frames.reshape(_PXROWS, 512)
    mask = jnp.asarray(_MASK2D)
    out = _sc_mask(f2, mask)
    return out.reshape(C, T, H, W)


# speedup vs baseline: 1.0014x; 1.0014x over previous
import functools

import jax
import jax.numpy as jnp
import numpy as np
from jax import lax
from jax.experimental import pallas as pl
from jax.experimental.pallas import tpu as pltpu
from jax.experimental.pallas import tpu_sc as plsc

_PATCH = 16
_SIDELEN = 32
_T = 16


def _threefry2x32(k1, k2, x0, x1):
    x0 = x0.astype(np.uint32).copy()
    x1 = x1.astype(np.uint32).copy()
    rot = ((13, 15, 26, 6), (17, 29, 16, 24))
    ks = (np.uint32(k1), np.uint32(k2),
          np.uint32(np.uint32(k1) ^ np.uint32(k2) ^ np.uint32(0x1BD11BDA)))
    x0 += ks[0]
    x1 += ks[1]
    for i in range(5):
        for r in rot[i % 2]:
            x0 += x1
            x1 = (x1 << np.uint32(r)) | (x1 >> np.uint32(32 - r))
            x1 ^= x0
        x0 += ks[(i + 1) % 3]
        x1 += ks[(i + 2) % 3] + np.uint32(i + 1)
    return x0, x1


def _np_split(key, num):
    b1, b2 = _threefry2x32(key[0], key[1],
                           np.zeros(num, np.uint32), np.arange(num, dtype=np.uint32))
    return np.stack([b1, b2], axis=1)


def _np_permutation(key, n):
    ks = _np_split(key, 2)
    b1, b2 = _threefry2x32(ks[1][0], ks[1][1],
                           np.zeros(n, np.uint32), np.arange(n, dtype=np.uint32))
    return np.argsort(b1 ^ b2, kind="stable").astype(np.int32)


def _build_row_mask() -> np.ndarray:
    num_patches = _SIDELEN * _SIDELEN
    num_masked = num_patches // 2
    keys = _np_split(np.array([0, 42], np.uint32), _T)
    idx = np.stack([_np_permutation(keys[t], num_patches)[:num_masked]
                    for t in range(_T)])
    h = idx % _SIDELEN
    w = idx // _SIDELEN
    pm = np.ones((_T, _SIDELEN, _SIDELEN), np.float32)
    pm[np.arange(_T)[:, None], h, w] = 0.0
    return np.repeat(pm, _PATCH, axis=2)


_ROW_MASK = _build_row_mask()
_MASK2D = np.tile(_ROW_MASK.reshape(_T * _SIDELEN, 512), (3, 1))

_NC, _NS = 2, 16
_NW = _NC * _NS
_BANDS = 3 * _T * _SIDELEN
_PXROWS = _BANDS * _PATCH
_BPW = _BANDS // _NW
_BCHUNK = 2
_PCHUNK = _BCHUNK * _PATCH
_NCHUNKS = _BPW // _BCHUNK
_NBUF = 4
_AHEAD = _NBUF - 1


def _sc_body(f_hbm, m_hbm, o_hbm, b0, b1, b2, b3, mbuf,
             si0, si1, si2, si3, so0, so1, so2, so3):
    bufs = (b0, b1, b2, b3)
    isems = (si0, si1, si2, si3)
    osems = (so0, so1, so2, so3)
    wid = lax.axis_index("s") * _NC + lax.axis_index("c")
    band0 = wid * _BPW
    px0 = band0 * _PATCH
    pltpu.sync_copy(m_hbm.at[pl.ds(band0, _BPW)], mbuf)

    for b in range(_AHEAD):
        pltpu.async_copy(f_hbm.at[pl.ds(px0 + b * _PCHUNK, _PCHUNK)],
                         bufs[b], isems[b])

    def compute_chunk(k, s):
        for r in range(_BCHUNK):
            mrow = k * _BCHUNK + r
            mvecs = [mbuf[mrow, pl.ds(v * 16, 16)] for v in range(_SIDELEN)]

            @plsc.parallel_loop(0, _PATCH, unroll=1)
            def px_body(py, _mvecs=mvecs, _r=r, _s=s):
                row = _r * _PATCH + py
                for v in range(_SIDELEN):
                    sl = (row, pl.ds(v * 16, 16))
                    bufs[_s][sl] = bufs[_s][sl] * _mvecs[v]

    def group_body(g, carry):
        for b in range(_NBUF):
            k = g * _NBUF + b
            s = b
            s2 = (b + _AHEAD) % _NBUF
            rs = px0 + k * _PCHUNK
            pltpu.make_async_copy(f_hbm.at[pl.ds(rs, _PCHUNK)],
                                  bufs[s], isems[s]).wait()
            compute_chunk(k, s)
            pltpu.async_copy(bufs[s], o_hbm.at[pl.ds(rs, _PCHUNK)], osems[s])

            @pl.when(k + _AHEAD < _NCHUNKS)
            def _(k=k, s2=s2):
                @pl.when(k >= 1)
                def _():
                    prs = px0 + (k - 1) * _PCHUNK
                    pltpu.make_async_copy(bufs[s2], o_hbm.at[pl.ds(prs, _PCHUNK)],
                                          osems[s2]).wait()

                nrs = px0 + (k + _AHEAD) * _PCHUNK
                pltpu.async_copy(f_hbm.at[pl.ds(nrs, _PCHUNK)], bufs[s2], isems[s2])
        return carry

    lax.fori_loop(0, _NCHUNKS // _NBUF, group_body, 0)

    for k in range(_NCHUNKS - _AHEAD - 1, _NCHUNKS):
        s = k % _NBUF
        lrs = px0 + k * _PCHUNK
        pltpu.make_async_copy(bufs[s], o_hbm.at[pl.ds(lrs, _PCHUNK)],
                              osems[s]).wait()


@functools.partial(
    pl.kernel,
    out_type=jax.ShapeDtypeStruct((_PXROWS, 512), jnp.float32),
    mesh=plsc.VectorSubcoreMesh(core_axis_name="c", subcore_axis_name="s"),
    scratch_types=[
        pltpu.VMEM((_PCHUNK, 512), jnp.float32),
        pltpu.VMEM((_PCHUNK, 512), jnp.float32),
        pltpu.VMEM((_PCHUNK, 512), jnp.float32),
        pltpu.VMEM((_PCHUNK, 512), jnp.float32),
        pltpu.VMEM((_BPW, 512), jnp.float32),
        pltpu.SemaphoreType.DMA,
        pltpu.SemaphoreType.DMA,
        pltpu.SemaphoreType.DMA,
        pltpu.SemaphoreType.DMA,
        pltpu.SemaphoreType.DMA,
        pltpu.SemaphoreType.DMA,
        pltpu.SemaphoreType.DMA,
        pltpu.SemaphoreType.DMA,
    ],
)
def _sc_mask(f_hbm, m_hbm, o_hbm, b0, b1, b2, b3, mbuf,
             si0, si1, si2, si3, so0, so1, so2, so3):
    _sc_body(f_hbm, m_hbm, o_hbm, b0, b1, b2, b3, mbuf,
             si0, si1, si2, si3, so0, so1, so2, so3)


def kernel(frames):
    C, T, H, W = frames.shape
    f2 = frames.reshape(_PXROWS, 512)
    mask = jnp.asarray(_MASK2D)
    out = _sc_mask(f2, mask)
    return out.reshape(C, T, H, W)
